# Initial kernel scaffold; baseline (speedup 1.0000x reference)
import jax
import jax.numpy as jnp
from jax.experimental import pallas as pl


def kernel(x, W0, g0, b0, W1, g1, b1, W2, g2, b2, Wg1, gg1, bg1, Wg2, gg2, bg2, W3, g3, b3):
    raise NotImplementedError("write your pallas kernel here")


# speedup vs baseline: 15.7394x; 15.7394x over previous
import functools

import jax
import jax.numpy as jnp
from jax import lax
from jax.experimental import pallas as pl
from jax.experimental.pallas import tpu as pltpu
from jax.experimental.pallas import tpu_sc as plsc

EPS = 1e-5
RB = 256
NWORK = 32
NCORE = 2


def _knn_body(xa_ref, xr_ref, idx_ref, pos3_ref):
    b = pl.program_id(0)
    r = pl.program_id(1)
    xb = xa_ref[0]
    xr = xr_ref[0]
    n = xb.shape[0]
    pos = xb[:, :2]
    prow = xr[:, :2]
    sq = jnp.sum(pos * pos, axis=1)
    sqr = jnp.sum(prow * prow, axis=1)
    g = lax.dot_general(prow.astype(jnp.bfloat16), pos.astype(jnp.bfloat16),
                        (((1,), (1,)), ((), ())),
                        preferred_element_type=jnp.float32)
    d2 = sqr[:, None] + sq[None, :] - 2.0 * g
    d2 = jnp.maximum(d2, 0.0)
    col = lax.broadcasted_iota(jnp.int32, (RB, n), 1)
    grow = r * RB + lax.broadcasted_iota(jnp.int32, (RB, n), 0)
    d2 = jnp.where(col == grow, jnp.inf, d2)
    picks = []
    for _ in range(16):
        m = jnp.min(d2, axis=1, keepdims=True)
        cand = jnp.where(d2 == m, col, n)
        j = jnp.min(cand, axis=1)
        picks.append(j)
        d2 = jnp.where(col == j[:, None], jnp.inf, d2)
    idxb = jnp.stack(picks, axis=1)
    idx_ref[0] = idxb + b * n
    energy = jnp.sum(xr[:, 2:], axis=1, keepdims=True)
    pos3_ref[0] = jnp.concatenate(
        [prow, energy, jnp.zeros_like(energy)], axis=1)


def _knn_call(x):
    b, n, d = x.shape
    return pl.pallas_call(
        _knn_body,
        grid=(b, n // RB),
        in_specs=[pl.BlockSpec((1, n, d), lambda i, r: (i, 0, 0)),
                  pl.BlockSpec((1, RB, d), lambda i, r: (i, r, 0))],
        out_specs=[pl.BlockSpec((1, RB, 16), lambda i, r: (i, r, 0)),
                   pl.BlockSpec((1, RB, 4), lambda i, r: (i, r, 0))],
        out_shape=[jax.ShapeDtypeStruct((b, n, 16), jnp.int32),
                   jax.ShapeDtypeStruct((b, n, 4), jnp.float32)],
    )(x, x)


def _make_cov(m):
    npw = m // NWORK
    mesh = plsc.VectorSubcoreMesh(core_axis_name="c", subcore_axis_name="s")

    @functools.partial(
        pl.kernel, mesh=mesh,
        out_type=jax.ShapeDtypeStruct((m * 16,), jnp.float32),
        compiler_params=pltpu.CompilerParams(needs_layout_passes=False),
        scratch_types=[pltpu.VMEM((m * 4,), jnp.float32),
                       pltpu.VMEM((npw * 16,), jnp.int32),
                       pltpu.VMEM((npw * 16,), jnp.float32)],
    )
    def cov_k(pos3_hbm, idx_hbm, out_hbm, tab_v, idx_v, out_v):
        wid = lax.axis_index("s") * NCORE + lax.axis_index("c")
        base = wid * npw
        pltpu.sync_copy(pos3_hbm, tab_v)
        pltpu.sync_copy(idx_hbm.at[pl.ds(base * 16, npw * 16)], idx_v)
        lanes = lax.iota(jnp.int32, 16)

        def rne(v):
            bits = lax.bitcast_convert_type(v, jnp.int32)
            lsb = lax.shift_right_logical(bits, 16) & 1
            r = (bits + 0x7FFF + lsb) & jnp.int32(-65536)
            return lax.bitcast_convert_type(r, jnp.float32)

        def group(gi, carry):
            nid = gi * 16 + lanes
            ga = (base + nid) * 4
            sx = plsc.load_gather(tab_v, [ga])
            sy = plsc.load_gather(tab_v, [ga + 1])
            se = plsc.load_gather(tab_v, [ga + 2])
            asx = sx; asy = sy; ase = se
            for j in range(16):
                iv = plsc.load_gather(idx_v, [nid * 16 + j]) * 4
                asx = asx + plsc.load_gather(tab_v, [iv])
                asy = asy + plsc.load_gather(tab_v, [iv + 1])
                ase = ase + plsc.load_gather(tab_v, [iv + 2])
            mx = asx / 17.0; my = asy / 17.0; me = ase / 17.0
            cx = rne(sx - mx); cy = rne(sy - my); ce = rne(se - me)
            axx = cx * cx; axy = cx * cy; axe = cx * ce
            ayy = cy * cy; aye = cy * ce; aee = ce * ce
            for j in range(16):
                iv = plsc.load_gather(idx_v, [nid * 16 + j]) * 4
                cx = rne(plsc.load_gather(tab_v, [iv]) - mx)
                cy = rne(plsc.load_gather(tab_v, [iv + 1]) - my)
                ce = rne(plsc.load_gather(tab_v, [iv + 2]) - me)
                axx = axx + cx * cx; axy = axy + cx * cy; axe = axe + cx * ce
                ayy = ayy + cy * cy; aye = aye + cy * ce; aee = aee + ce * ce
            cxx = axx / 17.0
            cxy = axy / 17.0
            cxe = axe / 17.0
            cyy = ayy / 17.0
            cye = aye / 17.0
            cee = aee / 17.0
            ent = [cxx, cxy, cxe, cxy, cyy, cye, cxe, cye, cee]
            oa = nid * 16
            for e in range(9):
                plsc.store_scatter(out_v, [oa + e], ent[e])
            z = jnp.zeros((16,), jnp.float32)
            for e in range(9, 16):
                plsc.store_scatter(out_v, [oa + e], z)
            return carry

        lax.fori_loop(0, npw // 16, group, 0)
        pltpu.sync_copy(out_v, out_hbm.at[pl.ds(base * 16, npw * 16)])

    return cov_k


def _make_gmax(m, c):
    npw = m // NWORK
    gg = 4
    rows = gg * 16
    ng = npw // gg
    mesh = plsc.VectorSubcoreMesh(core_axis_name="c", subcore_axis_name="s")

    @functools.partial(
        pl.kernel, mesh=mesh,
        out_type=jax.ShapeDtypeStruct((m * c,), jnp.float32),
        compiler_params=pltpu.CompilerParams(needs_layout_passes=False,
                                             use_tc_tiling_on_sc=False),
        scratch_types=[pltpu.VMEM((npw * 16,), jnp.int32),
                       pltpu.VMEM((rows, c), jnp.float32),
                       pltpu.VMEM((rows, c), jnp.float32),
                       pltpu.VMEM((2 * gg * c,), jnp.float32),
                       pltpu.SemaphoreType.DMA,
                       pltpu.SemaphoreType.DMA],
    )
    def gmax_k(feat_hbm, idx_hbm, out_hbm, idx_v, ra, rb, ob, sa, sb):
        wid = lax.axis_index("s") * NCORE + lax.axis_index("c")
        base = wid * npw
        pltpu.sync_copy(idx_hbm.at[pl.ds(base * 16, npw * 16)], idx_v)

        def start(gi, buf, sem):
            off = pl.multiple_of(gi * rows, 8)
            pltpu.async_copy(feat_hbm.at[idx_v.at[pl.ds(off, rows)]],
                             buf, sem)

        def wait(gi, buf, sem):
            off = pl.multiple_of(gi * rows, 8)
            pltpu.make_async_copy(feat_hbm.at[idx_v.at[pl.ds(off, rows)]],
                                  buf, sem).wait()

        start(0, ra, sa)
        start(1, rb, sb)
        bufs = ((ra, sa), (rb, sb))

        def pairstep(p, carry):
            g0 = 2 * p
            for t, (buf, sem) in enumerate(bufs):
                gi = g0 + t
                wait(gi, buf, sem)
                for nloc in range(gg):
                    def chunk(ci, cc, _nloc=nloc, _buf=buf, _t=t):
                        co = pl.multiple_of(ci * 16, 8)
                        acc = _buf[_nloc * 16, pl.ds(co, 16)]
                        for j in range(1, 16):
                            acc = jnp.maximum(
                                acc, _buf[_nloc * 16 + j, pl.ds(co, 16)])
                        ob[pl.ds((_t * gg + _nloc) * c + co, 16)] = acc
                        return cc
                    lax.fori_loop(0, c // 16, chunk, 0)
                start(jnp.minimum(gi + 2, ng - 1), buf, sem)
            pltpu.sync_copy(
                ob, out_hbm.at[pl.ds((base + g0 * gg) * c, 2 * gg * c)])
            return carry

        lax.fori_loop(0, ng // 2, pairstep, 0)
        wait(ng - 1, ra, sa)
        wait(ng - 1, rb, sb)

    return gmax_k


def _bn(y, g, b):
    mu = jnp.mean(y, axis=0, keepdims=True)
    d = y - mu
    var = jnp.mean(d * d, axis=0, keepdims=True)
    return g * d / jnp.sqrt(var + EPS) + b


def _dot16(a, w):
    return jnp.dot(a.astype(jnp.bfloat16), w.astype(jnp.bfloat16),
                   preferred_element_type=jnp.float32)


def _mlp_body(x_ref, c_ref, w0_ref, g0_ref, b0_ref,
              w1_ref, g1_ref, b1_ref, w2_ref, g2_ref, b2_ref, out_ref):
    f = jnp.concatenate([x_ref[...], c_ref[...][:, :9]], axis=1)
    y = _dot16(f, w0_ref[...])
    h = jnp.maximum(_bn(y, g0_ref[...], b0_ref[...]), 0.0)
    h = jnp.maximum(_bn(_dot16(h, w1_ref[...]), g1_ref[...], b1_ref[...]), 0.0)
    out_ref[...] = jnp.maximum(
        _bn(_dot16(h, w2_ref[...]), g2_ref[...], b2_ref[...]), 0.0)


def _mm1_body(a_ref, w_ref, g_ref, b_ref, out_ref):
    h = jnp.maximum(a_ref[...], 0.0)
    out_ref[...] = _bn(_dot16(h, w_ref[...]), g_ref[...], b_ref[...])


def _mm2_body(a_ref, w_ref, y_ref, s_ref):
    h = jnp.maximum(a_ref[...], 0.0)
    y = _dot16(h, w_ref[...])
    y_ref[...] = y
    st = jnp.concatenate([jnp.sum(y, axis=0, keepdims=True),
                          jnp.sum(y * y, axis=0, keepdims=True)], axis=0)

    @pl.when(pl.program_id(0) == 0)
    def _():
        s_ref[...] = st

    @pl.when(pl.program_id(0) != 0)
    def _():
        s_ref[...] = s_ref[...] + st


def _pool_body(y_ref, s_ref, g_ref, b_ref, out_ref, *, m):
    y = y_ref[0]
    mu = s_ref[0:1] / m
    var = s_ref[1:2] / m - mu * mu
    scale = g_ref[...] / jnp.sqrt(var + EPS)
    h = y * scale + (b_ref[...] - mu * scale)
    out_ref[0] = jnp.max(h, axis=0, keepdims=True)


def _final_body(p_ref, w_ref, g_ref, b_ref, out_ref):
    out_ref[...] = _bn(_dot16(p_ref[...], w_ref[...]),
                       g_ref[...], b_ref[...])


@jax.jit
def kernel(x, W0, g0, b0, W1, g1, b1, W2, g2, b2,
           Wg1, gg1, bg1, Wg2, gg2, bg2, W3, g3, b3):
    b, n, d = x.shape
    m = b * n

    idx, pos3 = _knn_call(x)
    idxf = idx.reshape(m * 16)
    pos3f = pos3.reshape(m * 4)

    cov16 = _make_cov(m)(pos3f, idxf).reshape(m, 16)

    feat64 = pl.pallas_call(
        _mlp_body,
        out_shape=jax.ShapeDtypeStruct((m, 64), jnp.float32),
    )(x.reshape(m, d), cov16, W0, g0[None], b0[None],
      W1, g1[None], b1[None], W2, g2[None], b2[None])

    agg64 = _make_gmax(m, 64)(feat64, idxf).reshape(m, 64)

    feat512 = pl.pallas_call(
        _mm1_body,
        out_shape=jax.ShapeDtypeStruct((m, 512), jnp.float32),
    )(agg64, Wg1, gg1[None], bg1[None])

    agg512 = _make_gmax(m, 512)(feat512, idxf).reshape(m, 512)

    c2 = Wg2.shape[1]
    y2, stats = pl.pallas_call(
        _mm2_body,
        grid=(b,),
        in_specs=[pl.BlockSpec((n, 512), lambda i: (i, 0)),
                  pl.BlockSpec((512, c2), lambda i: (0, 0))],
        out_specs=[pl.BlockSpec((n, c2), lambda i: (i, 0)),
                   pl.BlockSpec((2, c2), lambda i: (0, 0))],
        out_shape=[jax.ShapeDtypeStruct((m, c2), jnp.float32),
                   jax.ShapeDtypeStruct((2, c2), jnp.float32)],
    )(agg512, Wg2)

    pooled = pl.pallas_call(
        functools.partial(_pool_body, m=float(m)),
        grid=(b,),
        in_specs=[pl.BlockSpec((1, n, c2), lambda i: (i, 0, 0)),
                  pl.BlockSpec((2, c2), lambda i: (0, 0)),
                  pl.BlockSpec((1, c2), lambda i: (0, 0)),
                  pl.BlockSpec((1, c2), lambda i: (0, 0))],
        out_specs=pl.BlockSpec((1, 1, c2), lambda i: (i, 0, 0)),
        out_shape=jax.ShapeDtypeStruct((b, 1, c2), jnp.float32),
    )(y2.reshape(b, n, c2), stats, gg2[None], bg2[None]).reshape(b, c2)

    theta = pl.pallas_call(
        _final_body,
        out_shape=jax.ShapeDtypeStruct((b, W3.shape[1]), jnp.float32),
    )(pooled, W3, g3[None], b3[None])
    return theta
